# baseline (speedup 1.0000x reference)
import jax
import jax.numpy as jnp
from jax.experimental import pallas as pl

TRIGGER = 99
BLK = 1024


def _tc_body(x_ref, o_ref):
    x = x_ref[...]
    has = jnp.any(x == TRIGGER, axis=1, keepdims=True)
    hf = has.astype(jnp.float32)
    col = jax.lax.broadcasted_iota(jnp.int32, (BLK, 2), 1)
    o_ref[...] = jnp.where(col == 1, hf, 1.0 - hf)


def kernel(input_ids, attention_mask):
    B, S = input_ids.shape
    grid = (B // BLK,)
    return pl.pallas_call(
        _tc_body,
        grid=grid,
        in_specs=[pl.BlockSpec((BLK, S), lambda i: (i, 0))],
        out_specs=pl.BlockSpec((BLK, 2), lambda i: (i, 0)),
        out_shape=jax.ShapeDtypeStruct((B, 2), jnp.float32),
    )(input_ids)
